# Initial kernel scaffold; baseline (speedup 1.0000x reference)
import jax
import jax.numpy as jnp
from jax.experimental import pallas as pl


def kernel(token_ids, embedding):
    raise NotImplementedError("write your pallas kernel here")


# speedup vs baseline: 1.7447x; 1.7447x over previous
import jax
import jax.numpy as jnp
from jax.experimental import pallas as pl
from jax.experimental.pallas import tpu as pltpu
from jax.experimental.pallas import tpu_sc as plsc

EMBED_DIM = 64
WINDOW = 128


def kernel(token_ids, embedding):
    batch, hist = token_ids.shape
    n_idx = batch * hist
    idx = token_ids.reshape(1, n_idx).astype(jnp.int32)

    mesh = plsc.VectorSubcoreMesh(core_axis_name="core", subcore_axis_name="subcore")

    @pl.kernel(
        out_type=jax.ShapeDtypeStruct((n_idx, EMBED_DIM), embedding.dtype),
        mesh=mesh,
        compiler_params=pltpu.CompilerParams(use_tc_tiling_on_sc=False),
    )
    def gather_kernel(table_hbm, idx_hbm, out_hbm):
        def body(i_vmem, o_vmem):
            pltpu.sync_copy(table_hbm.at[i_vmem.at[0]], o_vmem)

        pltpu.emit_pipeline(
            body,
            grid=(n_idx // WINDOW,),
            in_specs=[pl.BlockSpec((1, WINDOW), index_map=lambda i: (0, i))],
            out_specs=[pl.BlockSpec((WINDOW, EMBED_DIM), index_map=lambda i: (i, 0))],
            core_axis_name=("core", "subcore"),
            dimension_semantics=(pltpu.PARALLEL,),
        )(idx_hbm, out_hbm)

    out = gather_kernel(embedding, idx)
    return out.reshape(batch, hist, EMBED_DIM)


# speedup vs baseline: 1.8719x; 1.0729x over previous
import jax
import jax.numpy as jnp
from jax.experimental import pallas as pl
from jax.experimental.pallas import tpu as pltpu
from jax.experimental.pallas import tpu_sc as plsc

EMBED_DIM = 64
WINDOW = 128
SUBGATHERS = 4


def kernel(token_ids, embedding):
    batch, hist = token_ids.shape
    n_idx = batch * hist
    step_rows = WINDOW * SUBGATHERS
    idx = token_ids.reshape(n_idx // WINDOW, WINDOW).astype(jnp.int32)

    mesh = plsc.VectorSubcoreMesh(core_axis_name="core", subcore_axis_name="subcore")

    @pl.kernel(
        out_type=jax.ShapeDtypeStruct((n_idx, EMBED_DIM), embedding.dtype),
        mesh=mesh,
        scratch_types=[pltpu.SemaphoreType.DMA],
        compiler_params=pltpu.CompilerParams(use_tc_tiling_on_sc=False),
    )
    def gather_kernel(table_hbm, idx_hbm, out_hbm, sem):
        def body(i_vmem, o_vmem):
            copies = [
                pltpu.async_copy(
                    table_hbm.at[i_vmem.at[j]],
                    o_vmem.at[pl.ds(j * WINDOW, WINDOW)],
                    sem,
                )
                for j in range(SUBGATHERS)
            ]
            for c in copies:
                c.wait()

        pltpu.emit_pipeline(
            body,
            grid=(n_idx // step_rows,),
            in_specs=[pl.BlockSpec((SUBGATHERS, WINDOW), index_map=lambda i: (i, 0))],
            out_specs=[pl.BlockSpec((step_rows, EMBED_DIM), index_map=lambda i: (i, 0))],
            core_axis_name=("core", "subcore"),
            dimension_semantics=(pltpu.PARALLEL,),
        )(idx_hbm, out_hbm)

    out = gather_kernel(embedding, idx)
    return out.reshape(batch, hist, EMBED_DIM)
